# baseline (speedup 1.0000x reference)
import functools

import jax
import jax.numpy as jnp
from jax import lax
from jax.experimental import pallas as pl
from jax.experimental.pallas import tpu as pltpu
from jax.experimental.pallas import tpu_sc as plsc

NC = 2
NS = 16
NW = NC * NS
LANES = 16


def _build_sc_scores(V, D, B, K):
    CB = B // NW
    C = 16
    NG = CB // C
    NIDX = C * K
    GSLICE = 80
    NGATHER = NIDX // GSLICE
    SCORES = C * (K + 1)
    NCH = D // LANES
    NT = K + 1

    mesh = plsc.VectorSubcoreMesh(core_axis_name="c", subcore_axis_name="s")

    @functools.partial(
        pl.kernel,
        mesh=mesh,
        out_type=jax.ShapeDtypeStruct((NW * LANES,), jnp.float32),
        scratch_types=[
            pltpu.VMEM((CB,), jnp.int32),
            pltpu.VMEM((CB,), jnp.int32),
            pltpu.VMEM((CB, K), jnp.int32),
            [pltpu.VMEM((NIDX,), jnp.int32) for _ in range(2)],
            [pltpu.VMEM((C + 1, D), jnp.float32) for _ in range(2)],
            [pltpu.VMEM((C + 1, D), jnp.float32) for _ in range(2)],
            [pltpu.VMEM((NIDX, D), jnp.float32) for _ in range(2)],
            pltpu.VMEM((SCORES, LANES), jnp.float32),
            pltpu.VMEM((LANES,), jnp.float32),
            [pltpu.SemaphoreType.DMA for _ in range(2)],
        ],
        compiler_params=pltpu.CompilerParams(
            needs_layout_passes=False, use_tc_tiling_on_sc=False
        ),
    )
    def sc_scores(syn0_h, syn1_h, pos_u_h, pos_v_h, negv_h, out_h,
                  uidx, vidx, nidx, nflat, u_rows, v_rows, n_rows, partial,
                  tacc, sem):
        cid = lax.axis_index("c")
        sid = lax.axis_index("s")
        wid = sid * NC + cid

        b0 = pl.multiple_of(wid * CB, CB)
        pltpu.sync_copy(pos_u_h.at[pl.ds(b0, CB)], uidx)
        pltpu.sync_copy(pos_v_h.at[pl.ds(b0, CB)], vidx)
        pltpu.sync_copy(negv_h.at[pl.ds(b0, CB)], nidx)

        def issue(g, p):
            l0 = pl.multiple_of(g * C, C)
            for i in range(C):
                nflat[p][pl.ds(i * K, LANES)] = \
                    nidx[l0 + i, pl.ds(0, LANES)]
                nflat[p][pl.ds(i * K + K - LANES, LANES)] = \
                    nidx[l0 + i, pl.ds(K - LANES, LANES)]
            pltpu.async_copy(syn0_h.at[uidx.at[pl.ds(l0, C)]],
                             u_rows[p].at[pl.ds(0, C)], sem[p])
            pltpu.async_copy(syn1_h.at[vidx.at[pl.ds(l0, C)]],
                             v_rows[p].at[pl.ds(0, C)], sem[p])
            for j in range(NGATHER):
                pltpu.async_copy(
                    syn1_h.at[nflat[p].at[pl.ds(j * GSLICE, GSLICE)]],
                    n_rows[p].at[pl.ds(j * GSLICE, GSLICE)],
                    sem[p],
                )

        def wait(p):
            pltpu.make_async_copy(syn0_h.at[uidx.at[pl.ds(0, C)]],
                                  u_rows[p].at[pl.ds(0, C)], sem[p]).wait()
            pltpu.make_async_copy(syn1_h.at[vidx.at[pl.ds(0, C)]],
                                  v_rows[p].at[pl.ds(0, C)], sem[p]).wait()
            for j in range(NGATHER):
                pltpu.make_async_copy(
                    syn1_h.at[nflat[p].at[pl.ds(j * GSLICE, GSLICE)]],
                    n_rows[p].at[pl.ds(j * GSLICE, GSLICE)],
                    sem[p],
                ).wait()

        iota16 = lax.iota(jnp.int32, LANES)
        cols = [jnp.full((LANES,), l, jnp.int32) for l in range(LANES)]
        tacc[...] = jnp.zeros((LANES,), jnp.float32)

        LN2_2P23 = 0.6931471805599453 / 8388608.0

        def _log_sigmoid16(x):
            z = jnp.exp(-jnp.abs(x))
            w = 1.0 + z
            bits = lax.bitcast_convert_type(w, jnp.int32) - 0x3F800000
            y = bits.astype(jnp.float32) * LN2_2P23
            for _ in range(3):
                y = y - 1.0 + w * jnp.exp(-y)
            return jnp.minimum(x, 0.0) - y

        def compute(g, p):
            ub, vb, nb = u_rows[p], v_rows[p], n_rows[p]

            def _dot(u, row_ref, r):
                prods = [
                    u[c] * row_ref[r, pl.ds(c * LANES, LANES)]
                    for c in range(NCH)
                ]
                while len(prods) > 1:
                    prods = [prods[2 * j] + prods[2 * j + 1]
                             for j in range(len(prods) // 2)]
                return prods[0]

            @plsc.parallel_loop(0, C, 1, unroll=2)
            def b_body(i):
                u = [ub[i, pl.ds(c * LANES, LANES)] for c in range(NCH)]
                p0 = i * NT
                r = i * K

                def _loads(t):
                    if t == 0:
                        return [vb[i, pl.ds(c * LANES, LANES)]
                                for c in range(NCH)]
                    return [nb[r + t - 1, pl.ds(c * LANES, LANES)]
                            for c in range(NCH)]

                def _tree(t, rows):
                    prods = [u[c] * rows[c] for c in range(NCH)]
                    while len(prods) > 1:
                        prods = [prods[2 * j] + prods[2 * j + 1]
                                 for j in range(len(prods) // 2)]
                    partial[p0 + t] = prods[0] if t == 0 else -prods[0]

                pend0 = _loads(0)
                pend1 = _loads(1)
                for t in range(NT):
                    nxt = _loads(t + 2) if t + 2 < NT else None
                    _tree(t, pend0)
                    pend0, pend1 = pend1, nxt

            zero16 = jnp.zeros((LANES,), jnp.float32)

            @plsc.parallel_loop(0, SCORES // LANES, 1, unroll=2,
                                carry=zero16)
            def r_body(g2, acc):
                rows = g2 * LANES + iota16
                vs = [
                    plsc.load_gather(partial, [rows, cols[l]])
                    for l in range(LANES)
                ]
                while len(vs) > 1:
                    vs = [vs[2 * j] + vs[2 * j + 1]
                          for j in range(len(vs) // 2)]
                return acc + _log_sigmoid16(vs[0])

            tacc[...] = tacc[...] + r_body

        issue(0, 0)

        def pair_body(j, carry):
            for p in range(2):
                g = j * 2 + p
                wait(p)
                gn = g + 1

                @pl.when(gn < NG)
                def _():
                    issue(gn, 1 - p)

                compute(g, p)
            return carry

        lax.fori_loop(0, NG // 2, pair_body, 0)

        pltpu.sync_copy(tacc, out_h.at[pl.ds(wid * LANES, LANES)])

    return sc_scores


def kernel(syn0, syn1, pos_u, pos_v, neg_v):
    V, D = syn0.shape
    B, K = neg_v.shape
    pos_u = pos_u.astype(jnp.int32)
    pos_v = pos_v.astype(jnp.int32)
    neg_v = neg_v.astype(jnp.int32)

    sc_scores = _build_sc_scores(V, D, B, K)
    psums = sc_scores(syn0, syn1, pos_u, pos_v, neg_v)
    return -jnp.sum(psums) / B


# speedup vs baseline: 1.0878x; 1.0878x over previous
import functools

import jax
import jax.numpy as jnp
from jax import lax
from jax.experimental import pallas as pl
from jax.experimental.pallas import tpu as pltpu
from jax.experimental.pallas import tpu_sc as plsc

NC = 2
NS = 16
NW = NC * NS
LANES = 16


def _build_sc_scores(V, D, B, K):
    CB = B // NW
    C = 16
    NG = CB // C
    NIDX = C * K
    GSLICE = 80
    NGATHER = NIDX // GSLICE
    SCORES = C * (K + 1)
    NCH = D // LANES
    NT = K + 1

    mesh = plsc.VectorSubcoreMesh(core_axis_name="c", subcore_axis_name="s")

    @functools.partial(
        pl.kernel,
        mesh=mesh,
        out_type=jax.ShapeDtypeStruct((NW * LANES,), jnp.float32),
        scratch_types=[
            pltpu.VMEM((CB,), jnp.int32),
            pltpu.VMEM((CB,), jnp.int32),
            pltpu.VMEM((CB * K,), jnp.int32),
            [pltpu.VMEM((C + 1, D), jnp.float32) for _ in range(2)],
            [pltpu.VMEM((C + 1, D), jnp.float32) for _ in range(2)],
            [pltpu.VMEM((NIDX, D), jnp.float32) for _ in range(2)],
            pltpu.VMEM((SCORES, LANES), jnp.float32),
            pltpu.VMEM((LANES,), jnp.float32),
            [pltpu.SemaphoreType.DMA for _ in range(2)],
        ],
        compiler_params=pltpu.CompilerParams(
            needs_layout_passes=False, use_tc_tiling_on_sc=False
        ),
    )
    def sc_scores(syn0_h, syn1_h, pos_u_h, pos_v_h, negv1d_h, out_h,
                  uidx, vidx, nidx, u_rows, v_rows, n_rows, partial,
                  tacc, sem):
        cid = lax.axis_index("c")
        sid = lax.axis_index("s")
        wid = sid * NC + cid

        b0 = pl.multiple_of(wid * CB, CB)
        pltpu.sync_copy(pos_u_h.at[pl.ds(b0, CB)], uidx)
        pltpu.sync_copy(pos_v_h.at[pl.ds(b0, CB)], vidx)
        pltpu.sync_copy(negv1d_h.at[pl.ds(b0 * K, CB * K)], nidx)

        def issue(g, p):
            l0 = pl.multiple_of(g * C, C)
            n0 = pl.multiple_of(g * NIDX, NIDX)
            pltpu.async_copy(syn0_h.at[uidx.at[pl.ds(l0, C)]],
                             u_rows[p].at[pl.ds(0, C)], sem[p])
            pltpu.async_copy(syn1_h.at[vidx.at[pl.ds(l0, C)]],
                             v_rows[p].at[pl.ds(0, C)], sem[p])
            for j in range(NGATHER):
                pltpu.async_copy(
                    syn1_h.at[nidx.at[pl.ds(n0 + j * GSLICE, GSLICE)]],
                    n_rows[p].at[pl.ds(j * GSLICE, GSLICE)],
                    sem[p],
                )

        def wait(p):
            pltpu.make_async_copy(syn0_h.at[uidx.at[pl.ds(0, C)]],
                                  u_rows[p].at[pl.ds(0, C)], sem[p]).wait()
            pltpu.make_async_copy(syn1_h.at[vidx.at[pl.ds(0, C)]],
                                  v_rows[p].at[pl.ds(0, C)], sem[p]).wait()
            for j in range(NGATHER):
                pltpu.make_async_copy(
                    syn1_h.at[nidx.at[pl.ds(j * GSLICE, GSLICE)]],
                    n_rows[p].at[pl.ds(j * GSLICE, GSLICE)],
                    sem[p],
                ).wait()

        iota16 = lax.iota(jnp.int32, LANES)
        cols = [jnp.full((LANES,), l, jnp.int32) for l in range(LANES)]
        tacc[...] = jnp.zeros((LANES,), jnp.float32)

        LN2_2P23 = 0.6931471805599453 / 8388608.0

        def _log_sigmoid16(x):
            z = jnp.exp(-jnp.abs(x))
            w = 1.0 + z
            bits = lax.bitcast_convert_type(w, jnp.int32) - 0x3F800000
            y = bits.astype(jnp.float32) * LN2_2P23
            for _ in range(3):
                y = y - 1.0 + w * jnp.exp(-y)
            return jnp.minimum(x, 0.0) - y

        def compute(g, p):
            ub, vb, nb = u_rows[p], v_rows[p], n_rows[p]

            def _dot(u, row_ref, r):
                prods = [
                    u[c] * row_ref[r, pl.ds(c * LANES, LANES)]
                    for c in range(NCH)
                ]
                while len(prods) > 1:
                    prods = [prods[2 * j] + prods[2 * j + 1]
                             for j in range(len(prods) // 2)]
                return prods[0]

            @plsc.parallel_loop(0, C, 1, unroll=2)
            def b_body(i):
                u = [ub[i, pl.ds(c * LANES, LANES)] for c in range(NCH)]
                p0 = i * NT
                r = i * K

                def _loads(t):
                    if t == 0:
                        return [vb[i, pl.ds(c * LANES, LANES)]
                                for c in range(NCH)]
                    return [nb[r + t - 1, pl.ds(c * LANES, LANES)]
                            for c in range(NCH)]

                def _tree(t, rows):
                    prods = [u[c] * rows[c] for c in range(NCH)]
                    while len(prods) > 1:
                        prods = [prods[2 * j] + prods[2 * j + 1]
                                 for j in range(len(prods) // 2)]
                    partial[p0 + t] = prods[0] if t == 0 else -prods[0]

                pend0 = _loads(0)
                pend1 = _loads(1)
                for t in range(NT):
                    nxt = _loads(t + 2) if t + 2 < NT else None
                    _tree(t, pend0)
                    pend0, pend1 = pend1, nxt

            zero16 = jnp.zeros((LANES,), jnp.float32)

            @plsc.parallel_loop(0, SCORES // LANES, 1, unroll=2,
                                carry=zero16)
            def r_body(g2, acc):
                rows = g2 * LANES + iota16
                vs = [
                    plsc.load_gather(partial, [rows, cols[l]])
                    for l in range(LANES)
                ]
                while len(vs) > 1:
                    vs = [vs[2 * j] + vs[2 * j + 1]
                          for j in range(len(vs) // 2)]
                return acc + _log_sigmoid16(vs[0])

            tacc[...] = tacc[...] + r_body

        issue(0, 0)

        def pair_body(j, carry):
            for p in range(2):
                g = j * 2 + p
                wait(p)
                gn = g + 1

                @pl.when(gn < NG)
                def _():
                    issue(gn, 1 - p)

                compute(g, p)
            return carry

        lax.fori_loop(0, NG // 2, pair_body, 0)

        pltpu.sync_copy(tacc, out_h.at[pl.ds(wid * LANES, LANES)])

    return sc_scores


def kernel(syn0, syn1, pos_u, pos_v, neg_v):
    V, D = syn0.shape
    B, K = neg_v.shape
    pos_u = pos_u.astype(jnp.int32)
    pos_v = pos_v.astype(jnp.int32)
    negv1d = neg_v.astype(jnp.int32).reshape(B * K)

    sc_scores = _build_sc_scores(V, D, B, K)
    psums = sc_scores(syn0, syn1, pos_u, pos_v, negv1d)
    return -jnp.sum(psums) / B
